# baseline (speedup 1.0000x reference)
import jax
import jax.numpy as jnp
from jax import lax
from jax.experimental import pallas as pl
from jax.experimental.pallas import tpu as pltpu
from jax.experimental.pallas import tpu_sc as plsc

N = 10000
E = 320000
D = 128
NT = 16
NP = 10240
RPT = NP // NT
EPT = E // NT
VPT = EPT // 16
EPB = EPT // 2
CHUNK = 80
NCHUNK = EPB // CHUNK
RB = 1000
NB = N // RB

_NEG_SLOPE = 0.2


def _node_body(x_ref, w_ref, as_ref, ad_ref,
               h_ref, s_ref, d_ref, m_ref, m_scr):
    i = pl.program_id(0)
    h = jnp.dot(x_ref[...], w_ref[...], preferred_element_type=jnp.float32)
    h_ref[...] = h
    s = jnp.dot(h, as_ref[...], preferred_element_type=jnp.float32)
    d = jnp.dot(h, ad_ref[...], preferred_element_type=jnp.float32)
    s_ref[...] = s
    d_ref[...] = d
    prev = jnp.where(i == 0, -jnp.inf, m_scr[0])
    m_scr[0] = jnp.maximum(prev, jnp.max(s))

    @pl.when(i == NB - 1)
    def _():
        m_ref[...] = jnp.full((1, 1), m_scr[0], jnp.float32)


def _node_project(x, W, a_s, a_d):
    din = x.shape[1]
    return pl.pallas_call(
        _node_body,
        grid=(NB,),
        in_specs=[
            pl.BlockSpec((RB, din), lambda i: (i, 0)),
            pl.BlockSpec((din, D), lambda i: (0, 0)),
            pl.BlockSpec((D, 1), lambda i: (0, 0)),
            pl.BlockSpec((D, 1), lambda i: (0, 0)),
        ],
        out_specs=[
            pl.BlockSpec((RB, D), lambda i: (i, 0)),
            pl.BlockSpec((RB, 1), lambda i: (i, 0)),
            pl.BlockSpec((RB, 1), lambda i: (i, 0)),
            pl.BlockSpec((1, 1), lambda i: (0, 0)),
        ],
        out_shape=[
            jax.ShapeDtypeStruct((N, D), jnp.float32),
            jax.ShapeDtypeStruct((N, 1), jnp.float32),
            jax.ShapeDtypeStruct((N, 1), jnp.float32),
            jax.ShapeDtypeStruct((1, 1), jnp.float32),
        ],
        scratch_shapes=[pltpu.SMEM((1,), jnp.float32)],
    )(x, W, a_s, a_d)


E8 = E // 8
EB = 4000
NEB = E8 // EB
DE = 16


RPK = 8000
NRPK = E // RPK


def _repack_body(ea_ref, out_ref):
    out_ref[...] = ea_ref[...].reshape(RPK // 8, D)


def _repack(edge_attr):
    return pl.pallas_call(
        _repack_body,
        grid=(NRPK,),
        in_specs=[pl.BlockSpec((RPK, DE), lambda i: (i, 0))],
        out_specs=pl.BlockSpec((RPK // 8, D), lambda i: (i, 0)),
        out_shape=jax.ShapeDtypeStruct((E8, D), jnp.float32),
    )(edge_attr)


def _edge_body(ea_ref, we1_ref, ae1_ref, we2_ref, ae2_ref,
               a1_ref, a2_ref, m1_ref, m2_ref, m_scr):
    i = pl.program_id(0)
    rows = lax.broadcasted_iota(jnp.int32, (D, 8), 0)
    cols = lax.broadcasted_iota(jnp.int32, (D, 8), 1)
    selr = lax.broadcasted_iota(jnp.int32, (D, DE), 0)
    selc = lax.broadcasted_iota(jnp.int32, (D, DE), 1)
    sel = jnp.where(jnp.equal(selr % DE, selc), 1.0, 0.0)
    bd = jnp.equal(rows // DE, cols)

    def vexp(we_ref, ae_ref):
        ve = jnp.dot(we_ref[...], ae_ref[...],
                     preferred_element_type=jnp.float32)
        vt = jnp.dot(sel, ve, preferred_element_type=jnp.float32)
        return jnp.where(bd, vt, 0.0)

    ea = ea_ref[...]
    a1 = jnp.dot(ea, vexp(we1_ref, ae1_ref),
                 preferred_element_type=jnp.float32)
    a2 = jnp.dot(ea, vexp(we2_ref, ae2_ref),
                 preferred_element_type=jnp.float32)
    a1_ref[...] = a1
    a2_ref[...] = a2
    p1 = jnp.where(i == 0, -jnp.inf, m_scr[0])
    p2 = jnp.where(i == 0, -jnp.inf, m_scr[1])
    m_scr[0] = jnp.maximum(p1, jnp.max(a1))
    m_scr[1] = jnp.maximum(p2, jnp.max(a2))

    @pl.when(i == NEB - 1)
    def _():
        m1_ref[...] = jnp.full((1, 1), m_scr[0], jnp.float32)
        m2_ref[...] = jnp.full((1, 1), m_scr[1], jnp.float32)


def _edge_logits(ea8, We1, ae1, We2, ae2):
    return pl.pallas_call(
        _edge_body,
        grid=(NEB,),
        in_specs=[
            pl.BlockSpec((EB, D), lambda i: (i, 0)),
            pl.BlockSpec((DE, D), lambda i: (0, 0)),
            pl.BlockSpec((D, 1), lambda i: (0, 0)),
            pl.BlockSpec((DE, D), lambda i: (0, 0)),
            pl.BlockSpec((D, 1), lambda i: (0, 0)),
        ],
        out_specs=[
            pl.BlockSpec((EB, 8), lambda i: (i, 0)),
            pl.BlockSpec((EB, 8), lambda i: (i, 0)),
            pl.BlockSpec((1, 1), lambda i: (0, 0)),
            pl.BlockSpec((1, 1), lambda i: (0, 0)),
        ],
        out_shape=[
            jax.ShapeDtypeStruct((E8, 8), jnp.float32),
            jax.ShapeDtypeStruct((E8, 8), jnp.float32),
            jax.ShapeDtypeStruct((1, 1), jnp.float32),
            jax.ShapeDtypeStruct((1, 1), jnp.float32),
        ],
        scratch_shapes=[pltpu.SMEM((2,), jnp.float32)],
    )(ea8, We1, ae1, We2, ae2)


def _mid_body(p0_ref, p1_ref, b_ref, w_ref, as_ref, ad_ref,
              hrelu_ref, h_ref, s_ref, d_ref, m_ref, m_scr):
    i = pl.program_id(0)
    xb = p0_ref[...] + p1_ref[...] + b_ref[...]
    xb = jnp.maximum(xb, 0.0)
    hrelu_ref[...] = xb
    h = jnp.dot(xb, w_ref[...], preferred_element_type=jnp.float32)
    h_ref[...] = h
    s = jnp.dot(h, as_ref[...], preferred_element_type=jnp.float32)
    d = jnp.dot(h, ad_ref[...], preferred_element_type=jnp.float32)
    s_ref[...] = s
    d_ref[...] = d
    prev = jnp.where(i == 0, -jnp.inf, m_scr[0])
    m_scr[0] = jnp.maximum(prev, jnp.max(s))

    @pl.when(i == NB - 1)
    def _():
        m_ref[...] = jnp.full((1, 1), m_scr[0], jnp.float32)


def _mid_project(p0, p1, b, W, a_s, a_d):
    return pl.pallas_call(
        _mid_body,
        grid=(NB,),
        in_specs=[
            pl.BlockSpec((RB, D), lambda i: (i, 0)),
            pl.BlockSpec((RB, D), lambda i: (i, 0)),
            pl.BlockSpec((1, D), lambda i: (0, 0)),
            pl.BlockSpec((D, D), lambda i: (0, 0)),
            pl.BlockSpec((D, 1), lambda i: (0, 0)),
            pl.BlockSpec((D, 1), lambda i: (0, 0)),
        ],
        out_specs=[
            pl.BlockSpec((RB, D), lambda i: (i, 0)),
            pl.BlockSpec((RB, D), lambda i: (i, 0)),
            pl.BlockSpec((RB, 1), lambda i: (i, 0)),
            pl.BlockSpec((RB, 1), lambda i: (i, 0)),
            pl.BlockSpec((1, 1), lambda i: (0, 0)),
        ],
        out_shape=[
            jax.ShapeDtypeStruct((N, D), jnp.float32),
            jax.ShapeDtypeStruct((N, D), jnp.float32),
            jax.ShapeDtypeStruct((N, 1), jnp.float32),
            jax.ShapeDtypeStruct((N, 1), jnp.float32),
            jax.ShapeDtypeStruct((1, 1), jnp.float32),
        ],
        scratch_shapes=[pltpu.SMEM((1,), jnp.float32)],
    )(p0, p1, b, W, a_s, a_d)


def _asm_body(p0_ref, p1_ref, b_ref, out_ref):
    out_ref[...] = p0_ref[...] + p1_ref[...] + b_ref[...]


def _assemble(p0, p1, b):
    return pl.pallas_call(
        _asm_body,
        grid=(NB,),
        in_specs=[
            pl.BlockSpec((RB, D), lambda i: (i, 0)),
            pl.BlockSpec((RB, D), lambda i: (i, 0)),
            pl.BlockSpec((1, D), lambda i: (0, 0)),
        ],
        out_specs=pl.BlockSpec((RB, D), lambda i: (i, 0)),
        out_shape=jax.ShapeDtypeStruct((N, D), jnp.float32),
    )(p0, p1, b)


BLK = 2000
VPB = BLK // 16
NBLK_A = EPT // BLK
NBLK_B = EPB // BLK
NSUB = BLK // CHUNK
NR = NP // D


def _rc(i16):
    return jnp.right_shift(i16, 7), jnp.bitwise_and(i16, 127)


def _ex16(v_asrc, v_adst, v_srcc, v_dstc, v_ae, it, scon):
    ds = pl.ds(it * 16, 16)
    s16 = v_srcc[ds]
    d16 = v_dstc[ds]
    ae16 = v_ae[ds]
    dr, dc = _rc(d16)
    asv = plsc.load_gather(v_asrc, list(_rc(s16)))
    adv = plsc.load_gather(v_adst, [dr, dc])
    tot = asv + adv + ae16
    al = jnp.maximum(tot, _NEG_SLOPE * tot)
    sh = adv + scon
    cshift = jnp.maximum(sh, _NEG_SLOPE * sh)
    return dr, dc, jnp.exp(al - cshift)


GRP = 5


def _sc_body(h_hbm, src_h, dst_h, aed_h, asrc_h, adst_h, scon_h,
             out0, out1, coef_h,
             v_t0, v_t1, v_t2, v_srcc, v_dstc, v_ae,
             v_d0, v_d1, v_d2, v_ri, v_sc,
             g0, g1, g2, s0, s1, s2,
             s_den, s_acc):
    cid = lax.axis_index("c")
    sid = lax.axis_index("s")
    ebase = sid * EPT
    rowbufs = [v_t0, v_t1, v_t2]
    dibufs = [v_d0, v_d1, v_d2]
    gsems = [g0, g1, g2]
    ssems = [s0, s1, s2]

    pltpu.sync_copy(scon_h, v_sc)
    scon = v_sc[pl.ds(0, 16)][0]

    def zrows(r, _):
        for j in range(D // 16):
            v_t0[r, pl.ds(j * 16, 16)] = jnp.zeros((16,), jnp.float32)
            v_t2[r, pl.ds(j * 16, 16)] = jnp.zeros((16,), jnp.float32)
        return 0
    lax.fori_loop(0, NR, zrows, 0)

    @pl.when(sid == 0)
    def _():
        pltpu.sync_copy(v_t0, s_den)
    for j in range(RPT // NR):
        pltpu.sync_copy(v_t0, s_acc.at[pl.ds(sid * RPT + j * NR, NR), :])

    pltpu.sync_copy(asrc_h, v_t0)
    pltpu.sync_copy(adst_h, v_t1)

    for j in range(NR // 16):
        v_ri[pl.ds(j * 16, 16)] = lax.iota(jnp.int32, 16) + (16 * j)

    def phase_a(blk, _):
        off = ebase + blk * BLK
        pltpu.sync_copy(src_h.at[pl.ds(off, BLK)], v_srcc)
        pltpu.sync_copy(dst_h.at[pl.ds(off, BLK)], v_dstc)
        pltpu.sync_copy(aed_h.at[pl.ds(off, BLK)], v_ae)

        def body(it, _):
            dr, dc, ex = _ex16(v_t0, v_t1, v_srcc, v_dstc, v_ae, it, scon)
            plsc.addupdate_scatter(v_t2, [dr, dc], ex)
            return 0
        lax.fori_loop(0, VPB, body, 0)
        return 0
    with jax.named_scope("phA"):
        lax.fori_loop(0, NBLK_A, phase_a, 0)

    with jax.named_scope("dred"):
        plsc.subcore_barrier()
        pltpu.sync_copy(v_t2, s_den.at[v_ri], add=True)
        plsc.subcore_barrier()
        pltpu.sync_copy(s_den, v_t2)

    def phase_coef(blk, _):
        off = ebase + cid * EPB + blk * BLK
        pltpu.sync_copy(src_h.at[pl.ds(off, BLK)], v_srcc)
        pltpu.sync_copy(dst_h.at[pl.ds(off, BLK)], v_dstc)
        pltpu.sync_copy(aed_h.at[pl.ds(off, BLK)], v_ae)

        def coef(it, _):
            dr, dc, ex = _ex16(v_t0, v_t1, v_srcc, v_dstc, v_ae, it, scon)
            den = plsc.load_gather(v_t2, [dr, dc])
            v_ae[pl.ds(it * 16, 16)] = ex / (den + 1e-16)
            return 0
        lax.fori_loop(0, VPB, coef, 0)
        pltpu.sync_copy(v_ae, coef_h.at[pl.ds(off, BLK)])
        return 0
    with jax.named_scope("phA5"):
        lax.fori_loop(0, NBLK_B, phase_coef, 0)

    def phase_b(blk, _):
        off = ebase + cid * EPB + blk * BLK
        pltpu.sync_copy(src_h.at[pl.ds(off, BLK)], v_srcc)
        pltpu.sync_copy(dst_h.at[pl.ds(off, BLK)], v_dstc)
        pltpu.sync_copy(coef_h.at[pl.ds(off, BLK)], v_ae)

        def group(g, _):
            gbase = g * (GRP * CHUNK)
            gd = [None] * GRP
            sd = [None] * GRP

            def fill_di(k):
                b = k % 3
                for j in range(CHUNK // 16):
                    dibufs[b][pl.ds(j * 16, 16)] = (
                        v_dstc[pl.ds(gbase + k * CHUNK + j * 16, 16)])

            def issue_gather(k):
                b = k % 3
                gd[k] = pltpu.async_copy(
                    h_hbm.at[v_srcc.at[pl.ds(gbase + k * CHUNK, CHUNK)]],
                    rowbufs[b], gsems[b])

            fill_di(0)
            issue_gather(0)
            fill_di(1)
            issue_gather(1)
            for k in range(GRP):
                b = k % 3
                gd[k].wait()

                def scale(r16, _, _b=b, _k=k):
                    cf16 = v_ae[pl.ds(gbase + _k * CHUNK + r16 * 16, 16)]
                    for jj in range(16):
                        cf = cf16[jj]
                        r = r16 * 16 + jj
                        for j in range(D // 16):
                            ds = pl.ds(j * 16, 16)
                            rowbufs[_b][r, ds] = rowbufs[_b][r, ds] * cf
                    return 0
                lax.fori_loop(0, CHUNK // 16, scale, 0)
                nk = k + 2
                if nk < GRP:
                    fill_di(nk)
                    issue_gather(nk)
            return 0
        lax.fori_loop(0, NSUB // GRP, group, 0)
        return 0
    with jax.named_scope("phB"):
        lax.fori_loop(0, NBLK_B, phase_b, 0)

    plsc.subcore_barrier()
    rds = pl.ds(sid * RPT, RPT)

    @pl.when(cid == 0)
    def _():
        pltpu.sync_copy(s_acc.at[rds], out0.at[rds])

    @pl.when(cid == 1)
    def _():
        pltpu.sync_copy(s_acc.at[rds], out1.at[rds])


_sc_layer = pl.kernel(
    _sc_body,
    out_type=(
        jax.ShapeDtypeStruct((NP, D), jnp.float32),
        jax.ShapeDtypeStruct((NP, D), jnp.float32),
        jax.ShapeDtypeStruct((E,), jnp.float32),
    ),
    mesh=plsc.VectorSubcoreMesh(core_axis_name="c", subcore_axis_name="s"),
    compiler_params=pltpu.CompilerParams(needs_layout_passes=False),
    scratch_types=dict(
        v_t0=pltpu.VMEM((NR, D), jnp.float32),
        v_t1=pltpu.VMEM((NR, D), jnp.float32),
        v_t2=pltpu.VMEM((NR, D), jnp.float32),
        v_srcc=pltpu.VMEM((BLK,), jnp.int32),
        v_dstc=pltpu.VMEM((BLK,), jnp.int32),
        v_ae=pltpu.VMEM((BLK,), jnp.float32),
        v_d0=pltpu.VMEM((CHUNK,), jnp.int32),
        v_d1=pltpu.VMEM((CHUNK,), jnp.int32),
        v_d2=pltpu.VMEM((CHUNK,), jnp.int32),
        v_ri=pltpu.VMEM((NR,), jnp.int32),
        v_sc=pltpu.VMEM((16,), jnp.float32),
        g0=pltpu.SemaphoreType.DMA,
        g1=pltpu.SemaphoreType.DMA,
        g2=pltpu.SemaphoreType.DMA,
        s0=pltpu.SemaphoreType.DMA,
        s1=pltpu.SemaphoreType.DMA,
        s2=pltpu.SemaphoreType.DMA,
        s_den=pltpu.VMEM_SHARED((NR, D), jnp.float32),
        s_acc=pltpu.VMEM_SHARED((NP, D), jnp.float32),
    ),
)


def _pad_nodes(a):
    return jnp.pad(a.reshape(N), (0, NP - N)).reshape(NR, D)


def kernel(x, edge_index, edge_attr, W1, We1, as1, ad1, ae1, b1,
           W2, We2, as2, ad2, ae2, b2):
    src = edge_index[0].astype(jnp.int32)
    dst = edge_index[1].astype(jnp.int32)

    h1, asrc1, adst1, m1 = _node_project(
        x, W1, as1.reshape(D, 1), ad1.reshape(D, 1))
    aed1, aed2, me1, me2 = _edge_logits(
        edge_attr.reshape(E8, D), We1, ae1.reshape(D, 1),
        We2, ae2.reshape(D, 1))

    scon1 = jnp.pad((m1 + me1).reshape(1), (0, 15))
    p1_0, p1_1, _ = _sc_layer(
        h1, src, dst, aed1.reshape(E),
        _pad_nodes(asrc1), _pad_nodes(adst1), scon1)

    h, h2, asrc2, adst2, m2 = _mid_project(
        p1_0[:N], p1_1[:N], b1.reshape(1, D), W2,
        as2.reshape(D, 1), ad2.reshape(D, 1))

    scon2 = jnp.pad((m2 + me2).reshape(1), (0, 15))
    p2_0, p2_1, _ = _sc_layer(
        h2, src, dst, aed2.reshape(E),
        _pad_nodes(asrc2), _pad_nodes(adst2), scon2)

    x2 = _assemble(p2_0[:N], p2_1[:N], b2.reshape(1, D))
    return (x2, h)


# speedup vs baseline: 1.0547x; 1.0547x over previous
import jax
import jax.numpy as jnp
from jax import lax
from jax.experimental import pallas as pl
from jax.experimental.pallas import tpu as pltpu
from jax.experimental.pallas import tpu_sc as plsc

N = 10000
E = 320000
D = 128
NT = 16
NP = 10240
RPT = NP // NT
EPT = E // NT
VPT = EPT // 16
EPB = EPT // 2
CHUNK = 80
NCHUNK = EPB // CHUNK
RB = 1000
NB = N // RB

_NEG_SLOPE = 0.2


def _node_body(x_ref, w_ref, as_ref, ad_ref,
               h_ref, s_ref, d_ref, m_ref, m_scr):
    i = pl.program_id(0)
    h = jnp.dot(x_ref[...], w_ref[...], preferred_element_type=jnp.float32)
    h_ref[...] = h
    s = jnp.dot(h, as_ref[...], preferred_element_type=jnp.float32)
    d = jnp.dot(h, ad_ref[...], preferred_element_type=jnp.float32)
    s_ref[...] = s
    d_ref[...] = d
    prev = jnp.where(i == 0, -jnp.inf, m_scr[0])
    m_scr[0] = jnp.maximum(prev, jnp.max(s))

    @pl.when(i == NB - 1)
    def _():
        m_ref[...] = jnp.full((1, 1), m_scr[0], jnp.float32)


def _node_project(x, W, a_s, a_d):
    din = x.shape[1]
    return pl.pallas_call(
        _node_body,
        grid=(NB,),
        in_specs=[
            pl.BlockSpec((RB, din), lambda i: (i, 0)),
            pl.BlockSpec((din, D), lambda i: (0, 0)),
            pl.BlockSpec((D, 1), lambda i: (0, 0)),
            pl.BlockSpec((D, 1), lambda i: (0, 0)),
        ],
        out_specs=[
            pl.BlockSpec((RB, D), lambda i: (i, 0)),
            pl.BlockSpec((RB, 1), lambda i: (i, 0)),
            pl.BlockSpec((RB, 1), lambda i: (i, 0)),
            pl.BlockSpec((1, 1), lambda i: (0, 0)),
        ],
        out_shape=[
            jax.ShapeDtypeStruct((N, D), jnp.float32),
            jax.ShapeDtypeStruct((N, 1), jnp.float32),
            jax.ShapeDtypeStruct((N, 1), jnp.float32),
            jax.ShapeDtypeStruct((1, 1), jnp.float32),
        ],
        scratch_shapes=[pltpu.SMEM((1,), jnp.float32)],
    )(x, W, a_s, a_d)


E8 = E // 8
EB = 4000
NEB = E8 // EB
DE = 16


RPK = 8000
NRPK = E // RPK


def _repack_body(ea_ref, out_ref):
    out_ref[...] = ea_ref[...].reshape(RPK // 8, D)


def _repack(edge_attr):
    return pl.pallas_call(
        _repack_body,
        grid=(NRPK,),
        in_specs=[pl.BlockSpec((RPK, DE), lambda i: (i, 0))],
        out_specs=pl.BlockSpec((RPK // 8, D), lambda i: (i, 0)),
        out_shape=jax.ShapeDtypeStruct((E8, D), jnp.float32),
    )(edge_attr)


def _edge_body(ea_ref, we1_ref, ae1_ref, we2_ref, ae2_ref,
               a1_ref, a2_ref, m1_ref, m2_ref, m_scr):
    i = pl.program_id(0)
    rows = lax.broadcasted_iota(jnp.int32, (D, 8), 0)
    cols = lax.broadcasted_iota(jnp.int32, (D, 8), 1)
    selr = lax.broadcasted_iota(jnp.int32, (D, DE), 0)
    selc = lax.broadcasted_iota(jnp.int32, (D, DE), 1)
    sel = jnp.where(jnp.equal(selr % DE, selc), 1.0, 0.0)
    bd = jnp.equal(rows // DE, cols)

    def vexp(we_ref, ae_ref):
        ve = jnp.dot(we_ref[...], ae_ref[...],
                     preferred_element_type=jnp.float32)
        vt = jnp.dot(sel, ve, preferred_element_type=jnp.float32)
        return jnp.where(bd, vt, 0.0)

    ea = ea_ref[...]
    a1 = jnp.dot(ea, vexp(we1_ref, ae1_ref),
                 preferred_element_type=jnp.float32)
    a2 = jnp.dot(ea, vexp(we2_ref, ae2_ref),
                 preferred_element_type=jnp.float32)
    a1_ref[...] = a1
    a2_ref[...] = a2
    p1 = jnp.where(i == 0, -jnp.inf, m_scr[0])
    p2 = jnp.where(i == 0, -jnp.inf, m_scr[1])
    m_scr[0] = jnp.maximum(p1, jnp.max(a1))
    m_scr[1] = jnp.maximum(p2, jnp.max(a2))

    @pl.when(i == NEB - 1)
    def _():
        m1_ref[...] = jnp.full((1, 1), m_scr[0], jnp.float32)
        m2_ref[...] = jnp.full((1, 1), m_scr[1], jnp.float32)


def _edge_logits(ea8, We1, ae1, We2, ae2):
    return pl.pallas_call(
        _edge_body,
        grid=(NEB,),
        in_specs=[
            pl.BlockSpec((EB, D), lambda i: (i, 0)),
            pl.BlockSpec((DE, D), lambda i: (0, 0)),
            pl.BlockSpec((D, 1), lambda i: (0, 0)),
            pl.BlockSpec((DE, D), lambda i: (0, 0)),
            pl.BlockSpec((D, 1), lambda i: (0, 0)),
        ],
        out_specs=[
            pl.BlockSpec((EB, 8), lambda i: (i, 0)),
            pl.BlockSpec((EB, 8), lambda i: (i, 0)),
            pl.BlockSpec((1, 1), lambda i: (0, 0)),
            pl.BlockSpec((1, 1), lambda i: (0, 0)),
        ],
        out_shape=[
            jax.ShapeDtypeStruct((E8, 8), jnp.float32),
            jax.ShapeDtypeStruct((E8, 8), jnp.float32),
            jax.ShapeDtypeStruct((1, 1), jnp.float32),
            jax.ShapeDtypeStruct((1, 1), jnp.float32),
        ],
        scratch_shapes=[pltpu.SMEM((2,), jnp.float32)],
    )(ea8, We1, ae1, We2, ae2)


def _mid_body(p0_ref, p1_ref, b_ref, w_ref, as_ref, ad_ref,
              hrelu_ref, h_ref, s_ref, d_ref, m_ref, m_scr):
    i = pl.program_id(0)
    xb = p0_ref[...] + p1_ref[...] + b_ref[...]
    xb = jnp.maximum(xb, 0.0)
    hrelu_ref[...] = xb
    h = jnp.dot(xb, w_ref[...], preferred_element_type=jnp.float32)
    h_ref[...] = h
    s = jnp.dot(h, as_ref[...], preferred_element_type=jnp.float32)
    d = jnp.dot(h, ad_ref[...], preferred_element_type=jnp.float32)
    s_ref[...] = s
    d_ref[...] = d
    prev = jnp.where(i == 0, -jnp.inf, m_scr[0])
    m_scr[0] = jnp.maximum(prev, jnp.max(s))

    @pl.when(i == NB - 1)
    def _():
        m_ref[...] = jnp.full((1, 1), m_scr[0], jnp.float32)


def _mid_project(p0, p1, b, W, a_s, a_d):
    return pl.pallas_call(
        _mid_body,
        grid=(NB,),
        in_specs=[
            pl.BlockSpec((RB, D), lambda i: (i, 0)),
            pl.BlockSpec((RB, D), lambda i: (i, 0)),
            pl.BlockSpec((1, D), lambda i: (0, 0)),
            pl.BlockSpec((D, D), lambda i: (0, 0)),
            pl.BlockSpec((D, 1), lambda i: (0, 0)),
            pl.BlockSpec((D, 1), lambda i: (0, 0)),
        ],
        out_specs=[
            pl.BlockSpec((RB, D), lambda i: (i, 0)),
            pl.BlockSpec((RB, D), lambda i: (i, 0)),
            pl.BlockSpec((RB, 1), lambda i: (i, 0)),
            pl.BlockSpec((RB, 1), lambda i: (i, 0)),
            pl.BlockSpec((1, 1), lambda i: (0, 0)),
        ],
        out_shape=[
            jax.ShapeDtypeStruct((N, D), jnp.float32),
            jax.ShapeDtypeStruct((N, D), jnp.float32),
            jax.ShapeDtypeStruct((N, 1), jnp.float32),
            jax.ShapeDtypeStruct((N, 1), jnp.float32),
            jax.ShapeDtypeStruct((1, 1), jnp.float32),
        ],
        scratch_shapes=[pltpu.SMEM((1,), jnp.float32)],
    )(p0, p1, b, W, a_s, a_d)


def _asm_body(p0_ref, p1_ref, b_ref, out_ref):
    out_ref[...] = p0_ref[...] + p1_ref[...] + b_ref[...]


def _assemble(p0, p1, b):
    return pl.pallas_call(
        _asm_body,
        grid=(NB,),
        in_specs=[
            pl.BlockSpec((RB, D), lambda i: (i, 0)),
            pl.BlockSpec((RB, D), lambda i: (i, 0)),
            pl.BlockSpec((1, D), lambda i: (0, 0)),
        ],
        out_specs=pl.BlockSpec((RB, D), lambda i: (i, 0)),
        out_shape=jax.ShapeDtypeStruct((N, D), jnp.float32),
    )(p0, p1, b)


BLK = 2000
VPB = BLK // 16
NBLK_A = EPT // BLK
NBLK_B = EPB // BLK
NSUB = BLK // CHUNK
NR = NP // D


def _rc(i16):
    return jnp.right_shift(i16, 7), jnp.bitwise_and(i16, 127)


def _ex16(v_asrc, v_adst, v_srcc, v_dstc, v_ae, it, scon):
    ds = pl.ds(it * 16, 16)
    s16 = v_srcc[ds]
    d16 = v_dstc[ds]
    ae16 = v_ae[ds]
    dr, dc = _rc(d16)
    asv = plsc.load_gather(v_asrc, list(_rc(s16)))
    adv = plsc.load_gather(v_adst, [dr, dc])
    tot = asv + adv + ae16
    al = jnp.maximum(tot, _NEG_SLOPE * tot)
    sh = adv + scon
    cshift = jnp.maximum(sh, _NEG_SLOPE * sh)
    return dr, dc, jnp.exp(al - cshift)


GRP = 5


def _sc_body(h_hbm, src_h, dst_h, aed_h, asrc_h, adst_h, scon_h,
             out0, out1, coef_h,
             v_t0, v_t1, v_t2, v_srcc, v_dstc, v_ae,
             v_srcb, v_dstb, v_aeb,
             v_d0, v_d1, v_d2, v_ri, v_sc,
             g0, g1, g2, s0, s1, s2, stA, stB, wb,
             s_den, s_acc):
    cid = lax.axis_index("c")
    sid = lax.axis_index("s")
    ebase = sid * EPT
    rowbufs = [v_t0, v_t1, v_t2]
    dibufs = [v_d0, v_d1, v_d2]
    gsems = [g0, g1, g2]
    ssems = [s0, s1, s2]
    sets = [(v_srcc, v_dstc, v_ae), (v_srcb, v_dstb, v_aeb)]
    stsems = [stA, stB]

    def stage(off, si, srcs):
        for i in range(3):
            pltpu.async_copy(srcs[i].at[pl.ds(off, BLK)],
                             sets[si][i], stsems[si])

    def wait_stage(si, srcs):
        for i in range(3):
            pltpu.make_async_copy(srcs[i].at[pl.ds(0, BLK)],
                                  sets[si][i], stsems[si]).wait()

    pltpu.sync_copy(scon_h, v_sc)
    scon = v_sc[pl.ds(0, 16)][0]

    def zrows(r, _):
        for j in range(D // 16):
            v_t0[r, pl.ds(j * 16, 16)] = jnp.zeros((16,), jnp.float32)
            v_t2[r, pl.ds(j * 16, 16)] = jnp.zeros((16,), jnp.float32)
        return 0
    lax.fori_loop(0, NR, zrows, 0)

    @pl.when(sid == 0)
    def _():
        pltpu.sync_copy(v_t0, s_den)
    for j in range(RPT // NR):
        pltpu.sync_copy(v_t0, s_acc.at[pl.ds(sid * RPT + j * NR, NR), :])

    pltpu.sync_copy(asrc_h, v_t0)
    pltpu.sync_copy(adst_h, v_t1)

    for j in range(NR // 16):
        v_ri[pl.ds(j * 16, 16)] = lax.iota(jnp.int32, 16) + (16 * j)

    srcs_a = (src_h, dst_h, aed_h)

    def a_compute(cur):
        def body(it, _):
            dr, dc, ex = _ex16(v_t0, v_t1, cur[0], cur[1], cur[2], it, scon)
            plsc.addupdate_scatter(v_t2, [dr, dc], ex)
            return 0
        lax.fori_loop(0, VPB, body, 0)

    with jax.named_scope("phA"):
        stage(ebase, 0, srcs_a)

        def a_pair(j, _):
            wait_stage(0, srcs_a)
            stage(ebase + (2 * j + 1) * BLK, 1, srcs_a)
            a_compute(sets[0])
            wait_stage(1, srcs_a)
            off2 = ebase + jnp.where(j == NBLK_A // 2 - 1, 0,
                                     (2 * j + 2)) * BLK
            stage(off2, 0, srcs_a)
            a_compute(sets[1])
            return 0
        lax.fori_loop(0, NBLK_A // 2, a_pair, 0)
        wait_stage(0, srcs_a)

    with jax.named_scope("dred"):
        plsc.subcore_barrier()
        pltpu.sync_copy(v_t2, s_den.at[v_ri], add=True)
        plsc.subcore_barrier()
        pltpu.sync_copy(s_den, v_t2)

    bbase = ebase + cid * EPB

    def coef_compute(cur, off):
        def coef(it, _):
            dr, dc, ex = _ex16(v_t0, v_t1, cur[0], cur[1], cur[2], it, scon)
            den = plsc.load_gather(v_t2, [dr, dc])
            cur[2][pl.ds(it * 16, 16)] = ex / (den + 1e-16)
            return 0
        lax.fori_loop(0, VPB, coef, 0)
        pltpu.sync_copy(cur[2], coef_h.at[pl.ds(off, BLK)])

    with jax.named_scope("phA5"):
        stage(bbase, 0, srcs_a)

        def c_pair(j, _):
            off0 = bbase + 2 * j * BLK
            wait_stage(0, srcs_a)
            stage(off0 + BLK, 1, srcs_a)
            coef_compute(sets[0], off0)
            wait_stage(1, srcs_a)
            stage(off0 + 2 * BLK, 0, srcs_a)
            coef_compute(sets[1], off0 + BLK)
            return 0
        lax.fori_loop(0, NBLK_B // 2, c_pair, 0)
        wait_stage(0, srcs_a)
        coef_compute(sets[0], bbase + (NBLK_B - 1) * BLK)

    srcs_b = (src_h, dst_h, coef_h)

    def b_block(cur):
        def group(g, _, _cur=cur):
                gbase = g * (GRP * CHUNK)
                gd = [None] * GRP
                sd = [None] * GRP

                def fill_di(k):
                    b = k % 3
                    for j in range(CHUNK // 16):
                        dibufs[b][pl.ds(j * 16, 16)] = (
                            _cur[1][pl.ds(gbase + k * CHUNK + j * 16, 16)])

                def issue_gather(k):
                    b = k % 3
                    gd[k] = pltpu.async_copy(
                        h_hbm.at[_cur[0].at[pl.ds(gbase + k * CHUNK, CHUNK)]],
                        rowbufs[b], gsems[b])

                fill_di(0)
                issue_gather(0)
                fill_di(1)
                issue_gather(1)
                for k in range(GRP):
                    b = k % 3
                    gd[k].wait()

                    def scale(r16, _, _b=b, _k=k):
                        cf16 = _cur[2][pl.ds(gbase + _k * CHUNK + r16 * 16,
                                             16)]
                        for jj in range(16):
                            cf = cf16[jj]
                            r = r16 * 16 + jj
                            for j in range(D // 16):
                                ds = pl.ds(j * 16, 16)
                                rowbufs[_b][r, ds] = rowbufs[_b][r, ds] * cf
                        return 0
                    lax.fori_loop(0, CHUNK // 16, scale, 0)
                    sd[k] = pltpu.async_copy(
                        rowbufs[b], s_acc.at[dibufs[b]], ssems[b], add=True)
                    nk = k + 2
                    if nk < GRP:
                        if nk >= 3:
                            sd[nk - 3].wait()
                        fill_di(nk)
                        issue_gather(nk)
                for k in range(max(GRP - 3, 2), GRP):
                    sd[k].wait()
                return 0
        lax.fori_loop(0, NSUB // GRP, group, 0)

    with jax.named_scope("phB"):
        stage(bbase, 0, srcs_b)

        def b_pair(j, _):
            wait_stage(0, srcs_b)
            stage(bbase + (2 * j + 1) * BLK, 1, srcs_b)
            b_block(sets[0])
            wait_stage(1, srcs_b)
            stage(bbase + (2 * j + 2) * BLK, 0, srcs_b)
            b_block(sets[1])
            return 0
        lax.fori_loop(0, NBLK_B // 2, b_pair, 0)
        wait_stage(0, srcs_b)
        b_block(sets[0])

    plsc.subcore_barrier()
    rds = pl.ds(sid * RPT, RPT)

    @pl.when(cid == 0)
    def _():
        pltpu.sync_copy(s_acc.at[rds], out0.at[rds])

    @pl.when(cid == 1)
    def _():
        pltpu.sync_copy(s_acc.at[rds], out1.at[rds])


_sc_layer = pl.kernel(
    _sc_body,
    out_type=(
        jax.ShapeDtypeStruct((NP, D), jnp.float32),
        jax.ShapeDtypeStruct((NP, D), jnp.float32),
        jax.ShapeDtypeStruct((E,), jnp.float32),
    ),
    mesh=plsc.VectorSubcoreMesh(core_axis_name="c", subcore_axis_name="s"),
    compiler_params=pltpu.CompilerParams(needs_layout_passes=False),
    scratch_types=dict(
        v_t0=pltpu.VMEM((NR, D), jnp.float32),
        v_t1=pltpu.VMEM((NR, D), jnp.float32),
        v_t2=pltpu.VMEM((NR, D), jnp.float32),
        v_srcc=pltpu.VMEM((BLK,), jnp.int32),
        v_dstc=pltpu.VMEM((BLK,), jnp.int32),
        v_ae=pltpu.VMEM((BLK,), jnp.float32),
        v_srcb=pltpu.VMEM((BLK,), jnp.int32),
        v_dstb=pltpu.VMEM((BLK,), jnp.int32),
        v_aeb=pltpu.VMEM((BLK,), jnp.float32),
        v_d0=pltpu.VMEM((CHUNK,), jnp.int32),
        v_d1=pltpu.VMEM((CHUNK,), jnp.int32),
        v_d2=pltpu.VMEM((CHUNK,), jnp.int32),
        v_ri=pltpu.VMEM((NR,), jnp.int32),
        v_sc=pltpu.VMEM((16,), jnp.float32),
        g0=pltpu.SemaphoreType.DMA,
        g1=pltpu.SemaphoreType.DMA,
        g2=pltpu.SemaphoreType.DMA,
        s0=pltpu.SemaphoreType.DMA,
        s1=pltpu.SemaphoreType.DMA,
        s2=pltpu.SemaphoreType.DMA,
        stA=pltpu.SemaphoreType.DMA,
        stB=pltpu.SemaphoreType.DMA,
        wb=pltpu.SemaphoreType.DMA,
        s_den=pltpu.VMEM_SHARED((NR, D), jnp.float32),
        s_acc=pltpu.VMEM_SHARED((NP, D), jnp.float32),
    ),
)


def _pad_nodes(a):
    return jnp.pad(a.reshape(N), (0, NP - N)).reshape(NR, D)


def kernel(x, edge_index, edge_attr, W1, We1, as1, ad1, ae1, b1,
           W2, We2, as2, ad2, ae2, b2):
    src = edge_index[0].astype(jnp.int32)
    dst = edge_index[1].astype(jnp.int32)

    h1, asrc1, adst1, m1 = _node_project(
        x, W1, as1.reshape(D, 1), ad1.reshape(D, 1))
    aed1, aed2, me1, me2 = _edge_logits(
        edge_attr.reshape(E8, D), We1, ae1.reshape(D, 1),
        We2, ae2.reshape(D, 1))

    scon1 = jnp.pad((m1 + me1).reshape(1), (0, 15))
    p1_0, p1_1, _ = _sc_layer(
        h1, src, dst, aed1.reshape(E),
        _pad_nodes(asrc1), _pad_nodes(adst1), scon1)

    h, h2, asrc2, adst2, m2 = _mid_project(
        p1_0[:N], p1_1[:N], b1.reshape(1, D), W2,
        as2.reshape(D, 1), ad2.reshape(D, 1))

    scon2 = jnp.pad((m2 + me2).reshape(1), (0, 15))
    p2_0, p2_1, _ = _sc_layer(
        h2, src, dst, aed2.reshape(E),
        _pad_nodes(asrc2), _pad_nodes(adst2), scon2)

    x2 = _assemble(p2_0[:N], p2_1[:N], b2.reshape(1, D))
    return (x2, h)


# speedup vs baseline: 1.0679x; 1.0125x over previous
import jax
import jax.numpy as jnp
from jax import lax
from jax.experimental import pallas as pl
from jax.experimental.pallas import tpu as pltpu
from jax.experimental.pallas import tpu_sc as plsc

N = 10000
E = 320000
D = 128
NT = 16
NP = 10240
RPT = NP // NT
EPT = E // NT
VPT = EPT // 16
EPB = EPT // 2
CHUNK = 80
NCHUNK = EPB // CHUNK
RB = 1000
NB = N // RB

_NEG_SLOPE = 0.2


def _node_body(x_ref, w_ref, as_ref, ad_ref,
               h_ref, s_ref, d_ref, m_ref, m_scr):
    i = pl.program_id(0)
    h = jnp.dot(x_ref[...], w_ref[...], preferred_element_type=jnp.float32)
    h_ref[...] = h
    s = jnp.dot(h, as_ref[...], preferred_element_type=jnp.float32)
    d = jnp.dot(h, ad_ref[...], preferred_element_type=jnp.float32)
    s_ref[...] = s
    d_ref[...] = d
    prev = jnp.where(i == 0, -jnp.inf, m_scr[0])
    m_scr[0] = jnp.maximum(prev, jnp.max(s))

    @pl.when(i == NB - 1)
    def _():
        m_ref[...] = jnp.full((1, 1), m_scr[0], jnp.float32)


def _node_project(x, W, a_s, a_d):
    din = x.shape[1]
    return pl.pallas_call(
        _node_body,
        grid=(NB,),
        in_specs=[
            pl.BlockSpec((RB, din), lambda i: (i, 0)),
            pl.BlockSpec((din, D), lambda i: (0, 0)),
            pl.BlockSpec((D, 1), lambda i: (0, 0)),
            pl.BlockSpec((D, 1), lambda i: (0, 0)),
        ],
        out_specs=[
            pl.BlockSpec((RB, D), lambda i: (i, 0)),
            pl.BlockSpec((RB, 1), lambda i: (i, 0)),
            pl.BlockSpec((RB, 1), lambda i: (i, 0)),
            pl.BlockSpec((1, 1), lambda i: (0, 0)),
        ],
        out_shape=[
            jax.ShapeDtypeStruct((N, D), jnp.float32),
            jax.ShapeDtypeStruct((N, 1), jnp.float32),
            jax.ShapeDtypeStruct((N, 1), jnp.float32),
            jax.ShapeDtypeStruct((1, 1), jnp.float32),
        ],
        scratch_shapes=[pltpu.SMEM((1,), jnp.float32)],
    )(x, W, a_s, a_d)


E8 = E // 8
EB = 4000
NEB = E8 // EB
DE = 16


RPK = 8000
NRPK = E // RPK


def _repack_body(ea_ref, out_ref):
    out_ref[...] = ea_ref[...].reshape(RPK // 8, D)


def _repack(edge_attr):
    return pl.pallas_call(
        _repack_body,
        grid=(NRPK,),
        in_specs=[pl.BlockSpec((RPK, DE), lambda i: (i, 0))],
        out_specs=pl.BlockSpec((RPK // 8, D), lambda i: (i, 0)),
        out_shape=jax.ShapeDtypeStruct((E8, D), jnp.float32),
    )(edge_attr)


def _edge_body(ea_ref, we1_ref, ae1_ref, we2_ref, ae2_ref,
               a1_ref, a2_ref, m1_ref, m2_ref, m_scr):
    i = pl.program_id(0)
    rows = lax.broadcasted_iota(jnp.int32, (D, 8), 0)
    cols = lax.broadcasted_iota(jnp.int32, (D, 8), 1)
    selr = lax.broadcasted_iota(jnp.int32, (D, DE), 0)
    selc = lax.broadcasted_iota(jnp.int32, (D, DE), 1)
    sel = jnp.where(jnp.equal(selr % DE, selc), 1.0, 0.0)
    bd = jnp.equal(rows // DE, cols)

    def vexp(we_ref, ae_ref):
        ve = jnp.dot(we_ref[...], ae_ref[...],
                     preferred_element_type=jnp.float32)
        vt = jnp.dot(sel, ve, preferred_element_type=jnp.float32)
        return jnp.where(bd, vt, 0.0)

    ea = ea_ref[...]
    a1 = jnp.dot(ea, vexp(we1_ref, ae1_ref),
                 preferred_element_type=jnp.float32)
    a2 = jnp.dot(ea, vexp(we2_ref, ae2_ref),
                 preferred_element_type=jnp.float32)
    a1_ref[...] = a1
    a2_ref[...] = a2
    p1 = jnp.where(i == 0, -jnp.inf, m_scr[0])
    p2 = jnp.where(i == 0, -jnp.inf, m_scr[1])
    m_scr[0] = jnp.maximum(p1, jnp.max(a1))
    m_scr[1] = jnp.maximum(p2, jnp.max(a2))

    @pl.when(i == NEB - 1)
    def _():
        m1_ref[...] = jnp.full((1, 1), m_scr[0], jnp.float32)
        m2_ref[...] = jnp.full((1, 1), m_scr[1], jnp.float32)


def _edge_logits(ea8, We1, ae1, We2, ae2):
    return pl.pallas_call(
        _edge_body,
        grid=(NEB,),
        in_specs=[
            pl.BlockSpec((EB, D), lambda i: (i, 0)),
            pl.BlockSpec((DE, D), lambda i: (0, 0)),
            pl.BlockSpec((D, 1), lambda i: (0, 0)),
            pl.BlockSpec((DE, D), lambda i: (0, 0)),
            pl.BlockSpec((D, 1), lambda i: (0, 0)),
        ],
        out_specs=[
            pl.BlockSpec((EB, 8), lambda i: (i, 0)),
            pl.BlockSpec((EB, 8), lambda i: (i, 0)),
            pl.BlockSpec((1, 1), lambda i: (0, 0)),
            pl.BlockSpec((1, 1), lambda i: (0, 0)),
        ],
        out_shape=[
            jax.ShapeDtypeStruct((E8, 8), jnp.float32),
            jax.ShapeDtypeStruct((E8, 8), jnp.float32),
            jax.ShapeDtypeStruct((1, 1), jnp.float32),
            jax.ShapeDtypeStruct((1, 1), jnp.float32),
        ],
        scratch_shapes=[pltpu.SMEM((2,), jnp.float32)],
    )(ea8, We1, ae1, We2, ae2)


def _mid_body(p0_ref, p1_ref, b_ref, w_ref, as_ref, ad_ref,
              hrelu_ref, h_ref, s_ref, d_ref, m_ref, m_scr):
    i = pl.program_id(0)
    xb = p0_ref[...] + p1_ref[...] + b_ref[...]
    xb = jnp.maximum(xb, 0.0)
    hrelu_ref[...] = xb
    h = jnp.dot(xb, w_ref[...], preferred_element_type=jnp.float32)
    h_ref[...] = h
    s = jnp.dot(h, as_ref[...], preferred_element_type=jnp.float32)
    d = jnp.dot(h, ad_ref[...], preferred_element_type=jnp.float32)
    s_ref[...] = s
    d_ref[...] = d
    prev = jnp.where(i == 0, -jnp.inf, m_scr[0])
    m_scr[0] = jnp.maximum(prev, jnp.max(s))

    @pl.when(i == NB - 1)
    def _():
        m_ref[...] = jnp.full((1, 1), m_scr[0], jnp.float32)


def _mid_project(p0, p1, b, W, a_s, a_d):
    return pl.pallas_call(
        _mid_body,
        grid=(NB,),
        in_specs=[
            pl.BlockSpec((RB, D), lambda i: (i, 0)),
            pl.BlockSpec((RB, D), lambda i: (i, 0)),
            pl.BlockSpec((1, D), lambda i: (0, 0)),
            pl.BlockSpec((D, D), lambda i: (0, 0)),
            pl.BlockSpec((D, 1), lambda i: (0, 0)),
            pl.BlockSpec((D, 1), lambda i: (0, 0)),
        ],
        out_specs=[
            pl.BlockSpec((RB, D), lambda i: (i, 0)),
            pl.BlockSpec((RB, D), lambda i: (i, 0)),
            pl.BlockSpec((RB, 1), lambda i: (i, 0)),
            pl.BlockSpec((RB, 1), lambda i: (i, 0)),
            pl.BlockSpec((1, 1), lambda i: (0, 0)),
        ],
        out_shape=[
            jax.ShapeDtypeStruct((N, D), jnp.float32),
            jax.ShapeDtypeStruct((N, D), jnp.float32),
            jax.ShapeDtypeStruct((N, 1), jnp.float32),
            jax.ShapeDtypeStruct((N, 1), jnp.float32),
            jax.ShapeDtypeStruct((1, 1), jnp.float32),
        ],
        scratch_shapes=[pltpu.SMEM((1,), jnp.float32)],
    )(p0, p1, b, W, a_s, a_d)


def _asm_body(p0_ref, p1_ref, b_ref, out_ref):
    out_ref[...] = p0_ref[...] + p1_ref[...] + b_ref[...]


def _assemble(p0, p1, b):
    return pl.pallas_call(
        _asm_body,
        grid=(NB,),
        in_specs=[
            pl.BlockSpec((RB, D), lambda i: (i, 0)),
            pl.BlockSpec((RB, D), lambda i: (i, 0)),
            pl.BlockSpec((1, D), lambda i: (0, 0)),
        ],
        out_specs=pl.BlockSpec((RB, D), lambda i: (i, 0)),
        out_shape=jax.ShapeDtypeStruct((N, D), jnp.float32),
    )(p0, p1, b)


BLK = 2000
VPB = BLK // 16
NBLK_A = EPT // BLK
NBLK_B = EPB // BLK
NSUB = BLK // CHUNK
NR = NP // D


def _rc(i16):
    return jnp.right_shift(i16, 7), jnp.bitwise_and(i16, 127)


def _ex16(v_asrc, v_adst, v_srcc, v_dstc, v_ae, it, scon):
    ds = pl.ds(it * 16, 16)
    s16 = v_srcc[ds]
    d16 = v_dstc[ds]
    ae16 = v_ae[ds]
    dr, dc = _rc(d16)
    asv = plsc.load_gather(v_asrc, list(_rc(s16)))
    adv = plsc.load_gather(v_adst, [dr, dc])
    tot = asv + adv + ae16
    al = jnp.maximum(tot, _NEG_SLOPE * tot)
    sh = adv + scon
    cshift = jnp.maximum(sh, _NEG_SLOPE * sh)
    return dr, dc, jnp.exp(al - cshift)


GRP = 5


def _sc_body(h_hbm, src_h, dst_h, aed_h, asrc_h, adst_h, scon_h,
             out0, out1,
             v_t0, v_t1, v_t2, v_srcc, v_dstc, v_ae,
             v_srcb, v_dstb, v_aeb,
             v_d0, v_d1, v_d2, v_ri, v_sc,
             g0, g1, g2, s0, s1, s2, stA, stB, wb,
             s_den, s_acc):
    cid = lax.axis_index("c")
    sid = lax.axis_index("s")
    ebase = sid * EPT
    rowbufs = [v_t0, v_t1, v_t2]
    dibufs = [v_d0, v_d1, v_d2]
    gsems = [g0, g1, g2]
    ssems = [s0, s1, s2]
    sets = [(v_srcc, v_dstc, v_ae), (v_srcb, v_dstb, v_aeb)]
    stsems = [stA, stB]

    def stage(off, si, srcs):
        for i in range(3):
            pltpu.async_copy(srcs[i].at[pl.ds(off, BLK)],
                             sets[si][i], stsems[si])

    def wait_stage(si, srcs):
        for i in range(3):
            pltpu.make_async_copy(srcs[i].at[pl.ds(0, BLK)],
                                  sets[si][i], stsems[si]).wait()

    pltpu.sync_copy(scon_h, v_sc)
    scon = v_sc[pl.ds(0, 16)][0]

    def zrows(r, _):
        for j in range(D // 16):
            v_t0[r, pl.ds(j * 16, 16)] = jnp.zeros((16,), jnp.float32)
            v_t2[r, pl.ds(j * 16, 16)] = jnp.zeros((16,), jnp.float32)
        return 0
    lax.fori_loop(0, NR, zrows, 0)

    @pl.when(sid == 0)
    def _():
        pltpu.sync_copy(v_t0, s_den)
    for j in range(RPT // NR):
        pltpu.sync_copy(v_t0, s_acc.at[pl.ds(sid * RPT + j * NR, NR), :])

    pltpu.sync_copy(asrc_h, v_t0)
    pltpu.sync_copy(adst_h, v_t1)

    for j in range(NR // 16):
        v_ri[pl.ds(j * 16, 16)] = lax.iota(jnp.int32, 16) + (16 * j)

    srcs_a = (src_h, dst_h, aed_h)

    def a_compute(cur):
        def body(it, _):
            dr, dc, ex = _ex16(v_t0, v_t1, cur[0], cur[1], cur[2], it, scon)
            plsc.addupdate_scatter(v_t2, [dr, dc], ex)
            return 0
        lax.fori_loop(0, VPB, body, 0)

    with jax.named_scope("phA"):
        stage(ebase, 0, srcs_a)

        def a_pair(j, _):
            wait_stage(0, srcs_a)
            stage(ebase + (2 * j + 1) * BLK, 1, srcs_a)
            a_compute(sets[0])
            wait_stage(1, srcs_a)
            off2 = ebase + jnp.where(j == NBLK_A // 2 - 1, 0,
                                     (2 * j + 2)) * BLK
            stage(off2, 0, srcs_a)
            a_compute(sets[1])
            return 0
        lax.fori_loop(0, NBLK_A // 2, a_pair, 0)
        wait_stage(0, srcs_a)

    with jax.named_scope("dred"):
        plsc.subcore_barrier()
        pltpu.sync_copy(v_t2, s_den.at[v_ri], add=True)
        plsc.subcore_barrier()
        pltpu.sync_copy(s_den, v_t2)

    bbase = ebase + cid * EPB

    def coef_compute(cur):
        def coef(it, _):
            dr, dc, ex = _ex16(v_t0, v_t1, cur[0], cur[1], cur[2], it, scon)
            den = plsc.load_gather(v_t2, [dr, dc])
            cur[2][pl.ds(it * 16, 16)] = ex / (den + 1e-16)
            return 0
        lax.fori_loop(0, VPB, coef, 0)

    def b_block(cur):
        def group(g, _, _cur=cur):
                gbase = g * (GRP * CHUNK)
                gd = [None] * GRP
                sd = [None] * GRP

                def fill_di(k):
                    b = k % 3
                    for j in range(CHUNK // 16):
                        dibufs[b][pl.ds(j * 16, 16)] = (
                            _cur[1][pl.ds(gbase + k * CHUNK + j * 16, 16)])

                def issue_gather(k):
                    b = k % 3
                    gd[k] = pltpu.async_copy(
                        h_hbm.at[_cur[0].at[pl.ds(gbase + k * CHUNK, CHUNK)]],
                        rowbufs[b], gsems[b])

                fill_di(0)
                issue_gather(0)
                fill_di(1)
                issue_gather(1)
                for k in range(GRP):
                    b = k % 3
                    gd[k].wait()

                    def scale(r16, _, _b=b, _k=k):
                        cf16 = _cur[2][pl.ds(gbase + _k * CHUNK + r16 * 16,
                                             16)]
                        for jj in range(16):
                            cf = cf16[jj]
                            r = r16 * 16 + jj
                            for j in range(D // 16):
                                ds = pl.ds(j * 16, 16)
                                rowbufs[_b][r, ds] = rowbufs[_b][r, ds] * cf
                        return 0
                    lax.fori_loop(0, CHUNK // 16, scale, 0)
                    sd[k] = pltpu.async_copy(
                        rowbufs[b], s_acc.at[dibufs[b]], ssems[b], add=True)
                    nk = k + 2
                    if nk < GRP:
                        if nk >= 3:
                            sd[nk - 3].wait()
                        fill_di(nk)
                        issue_gather(nk)
                for k in range(max(GRP - 3, 2), GRP):
                    sd[k].wait()
                return 0
        lax.fori_loop(0, NSUB // GRP, group, 0)

    with jax.named_scope("phB"):
        stage(bbase, 0, srcs_a)

        def b_pair(j, _):
            wait_stage(0, srcs_a)
            stage(bbase + (2 * j + 1) * BLK, 1, srcs_a)
            coef_compute(sets[0])
            b_block(sets[0])
            wait_stage(1, srcs_a)
            stage(bbase + (2 * j + 2) * BLK, 0, srcs_a)
            coef_compute(sets[1])
            b_block(sets[1])
            return 0
        lax.fori_loop(0, NBLK_B // 2, b_pair, 0)
        wait_stage(0, srcs_a)
        coef_compute(sets[0])
        b_block(sets[0])

    plsc.subcore_barrier()
    rds = pl.ds(sid * RPT, RPT)

    @pl.when(cid == 0)
    def _():
        pltpu.sync_copy(s_acc.at[rds], out0.at[rds])

    @pl.when(cid == 1)
    def _():
        pltpu.sync_copy(s_acc.at[rds], out1.at[rds])


_sc_layer = pl.kernel(
    _sc_body,
    out_type=(
        jax.ShapeDtypeStruct((NP, D), jnp.float32),
        jax.ShapeDtypeStruct((NP, D), jnp.float32),
    ),
    mesh=plsc.VectorSubcoreMesh(core_axis_name="c", subcore_axis_name="s"),
    compiler_params=pltpu.CompilerParams(needs_layout_passes=False),
    scratch_types=dict(
        v_t0=pltpu.VMEM((NR, D), jnp.float32),
        v_t1=pltpu.VMEM((NR, D), jnp.float32),
        v_t2=pltpu.VMEM((NR, D), jnp.float32),
        v_srcc=pltpu.VMEM((BLK,), jnp.int32),
        v_dstc=pltpu.VMEM((BLK,), jnp.int32),
        v_ae=pltpu.VMEM((BLK,), jnp.float32),
        v_srcb=pltpu.VMEM((BLK,), jnp.int32),
        v_dstb=pltpu.VMEM((BLK,), jnp.int32),
        v_aeb=pltpu.VMEM((BLK,), jnp.float32),
        v_d0=pltpu.VMEM((CHUNK,), jnp.int32),
        v_d1=pltpu.VMEM((CHUNK,), jnp.int32),
        v_d2=pltpu.VMEM((CHUNK,), jnp.int32),
        v_ri=pltpu.VMEM((NR,), jnp.int32),
        v_sc=pltpu.VMEM((16,), jnp.float32),
        g0=pltpu.SemaphoreType.DMA,
        g1=pltpu.SemaphoreType.DMA,
        g2=pltpu.SemaphoreType.DMA,
        s0=pltpu.SemaphoreType.DMA,
        s1=pltpu.SemaphoreType.DMA,
        s2=pltpu.SemaphoreType.DMA,
        stA=pltpu.SemaphoreType.DMA,
        stB=pltpu.SemaphoreType.DMA,
        wb=pltpu.SemaphoreType.DMA,
        s_den=pltpu.VMEM_SHARED((NR, D), jnp.float32),
        s_acc=pltpu.VMEM_SHARED((NP, D), jnp.float32),
    ),
)


def _pad_nodes(a):
    return jnp.pad(a.reshape(N), (0, NP - N)).reshape(NR, D)


def kernel(x, edge_index, edge_attr, W1, We1, as1, ad1, ae1, b1,
           W2, We2, as2, ad2, ae2, b2):
    src = edge_index[0].astype(jnp.int32)
    dst = edge_index[1].astype(jnp.int32)

    h1, asrc1, adst1, m1 = _node_project(
        x, W1, as1.reshape(D, 1), ad1.reshape(D, 1))
    aed1, aed2, me1, me2 = _edge_logits(
        edge_attr.reshape(E8, D), We1, ae1.reshape(D, 1),
        We2, ae2.reshape(D, 1))

    scon1 = jnp.pad((m1 + me1).reshape(1), (0, 15))
    p1_0, p1_1 = _sc_layer(
        h1, src, dst, aed1.reshape(E),
        _pad_nodes(asrc1), _pad_nodes(adst1), scon1)

    h, h2, asrc2, adst2, m2 = _mid_project(
        p1_0[:N], p1_1[:N], b1.reshape(1, D), W2,
        as2.reshape(D, 1), ad2.reshape(D, 1))

    scon2 = jnp.pad((m2 + me2).reshape(1), (0, 15))
    p2_0, p2_1 = _sc_layer(
        h2, src, dst, aed2.reshape(E),
        _pad_nodes(asrc2), _pad_nodes(adst2), scon2)

    x2 = _assemble(p2_0[:N], p2_1[:N], b2.reshape(1, D))
    return (x2, h)


# speedup vs baseline: 1.0934x; 1.0239x over previous
import jax
import jax.numpy as jnp
from jax import lax
from jax.experimental import pallas as pl
from jax.experimental.pallas import tpu as pltpu
from jax.experimental.pallas import tpu_sc as plsc

N = 10000
E = 320000
D = 128
NT = 16
NP = 10240
RPT = NP // NT
EPT = E // NT
VPT = EPT // 16
EPB = EPT // 2
CHUNK = 80
NCHUNK = EPB // CHUNK
RB = 1000
NB = N // RB

_NEG_SLOPE = 0.2


def _node_body(x_ref, w_ref, as_ref, ad_ref,
               h_ref, s_ref, d_ref, m_ref, m_scr):
    i = pl.program_id(0)
    h = jnp.dot(x_ref[...], w_ref[...], preferred_element_type=jnp.float32)
    h_ref[...] = h
    s = jnp.dot(h, as_ref[...], preferred_element_type=jnp.float32)
    d = jnp.dot(h, ad_ref[...], preferred_element_type=jnp.float32)
    s_ref[...] = s
    d_ref[...] = d
    prev = jnp.where(i == 0, -jnp.inf, m_scr[0])
    m_scr[0] = jnp.maximum(prev, jnp.max(s))

    @pl.when(i == NB - 1)
    def _():
        m_ref[...] = jnp.full((1, 1), m_scr[0], jnp.float32)


def _node_project(x, W, a_s, a_d):
    din = x.shape[1]
    return pl.pallas_call(
        _node_body,
        grid=(NB,),
        in_specs=[
            pl.BlockSpec((RB, din), lambda i: (i, 0)),
            pl.BlockSpec((din, D), lambda i: (0, 0)),
            pl.BlockSpec((D, 1), lambda i: (0, 0)),
            pl.BlockSpec((D, 1), lambda i: (0, 0)),
        ],
        out_specs=[
            pl.BlockSpec((RB, D), lambda i: (i, 0)),
            pl.BlockSpec((RB, 1), lambda i: (i, 0)),
            pl.BlockSpec((RB, 1), lambda i: (i, 0)),
            pl.BlockSpec((1, 1), lambda i: (0, 0)),
        ],
        out_shape=[
            jax.ShapeDtypeStruct((N, D), jnp.float32),
            jax.ShapeDtypeStruct((N, 1), jnp.float32),
            jax.ShapeDtypeStruct((N, 1), jnp.float32),
            jax.ShapeDtypeStruct((1, 1), jnp.float32),
        ],
        scratch_shapes=[pltpu.SMEM((1,), jnp.float32)],
    )(x, W, a_s, a_d)


E8 = E // 8
EB = 4000
NEB = E8 // EB
DE = 16


RPK = 8000
NRPK = E // RPK


def _repack_body(ea_ref, out_ref):
    out_ref[...] = ea_ref[...].reshape(RPK // 8, D)


def _repack(edge_attr):
    return pl.pallas_call(
        _repack_body,
        grid=(NRPK,),
        in_specs=[pl.BlockSpec((RPK, DE), lambda i: (i, 0))],
        out_specs=pl.BlockSpec((RPK // 8, D), lambda i: (i, 0)),
        out_shape=jax.ShapeDtypeStruct((E8, D), jnp.float32),
    )(edge_attr)


def _edge_body(ea_ref, we1_ref, ae1_ref, we2_ref, ae2_ref,
               a1_ref, a2_ref, m1_ref, m2_ref, m_scr):
    i = pl.program_id(0)
    rows = lax.broadcasted_iota(jnp.int32, (D, 8), 0)
    cols = lax.broadcasted_iota(jnp.int32, (D, 8), 1)
    selr = lax.broadcasted_iota(jnp.int32, (D, DE), 0)
    selc = lax.broadcasted_iota(jnp.int32, (D, DE), 1)
    sel = jnp.where(jnp.equal(selr % DE, selc), 1.0, 0.0)
    bd = jnp.equal(rows // DE, cols)

    def vexp(we_ref, ae_ref):
        ve = jnp.dot(we_ref[...], ae_ref[...],
                     preferred_element_type=jnp.float32)
        vt = jnp.dot(sel, ve, preferred_element_type=jnp.float32)
        return jnp.where(bd, vt, 0.0)

    ea = ea_ref[...]
    a1 = jnp.dot(ea, vexp(we1_ref, ae1_ref),
                 preferred_element_type=jnp.float32)
    a2 = jnp.dot(ea, vexp(we2_ref, ae2_ref),
                 preferred_element_type=jnp.float32)
    a1_ref[...] = a1
    a2_ref[...] = a2
    p1 = jnp.where(i == 0, -jnp.inf, m_scr[0])
    p2 = jnp.where(i == 0, -jnp.inf, m_scr[1])
    m_scr[0] = jnp.maximum(p1, jnp.max(a1))
    m_scr[1] = jnp.maximum(p2, jnp.max(a2))

    @pl.when(i == NEB - 1)
    def _():
        m1_ref[...] = jnp.full((1, 1), m_scr[0], jnp.float32)
        m2_ref[...] = jnp.full((1, 1), m_scr[1], jnp.float32)


def _edge_logits(ea8, We1, ae1, We2, ae2):
    return pl.pallas_call(
        _edge_body,
        grid=(NEB,),
        in_specs=[
            pl.BlockSpec((EB, D), lambda i: (i, 0)),
            pl.BlockSpec((DE, D), lambda i: (0, 0)),
            pl.BlockSpec((D, 1), lambda i: (0, 0)),
            pl.BlockSpec((DE, D), lambda i: (0, 0)),
            pl.BlockSpec((D, 1), lambda i: (0, 0)),
        ],
        out_specs=[
            pl.BlockSpec((EB, 8), lambda i: (i, 0)),
            pl.BlockSpec((EB, 8), lambda i: (i, 0)),
            pl.BlockSpec((1, 1), lambda i: (0, 0)),
            pl.BlockSpec((1, 1), lambda i: (0, 0)),
        ],
        out_shape=[
            jax.ShapeDtypeStruct((E8, 8), jnp.float32),
            jax.ShapeDtypeStruct((E8, 8), jnp.float32),
            jax.ShapeDtypeStruct((1, 1), jnp.float32),
            jax.ShapeDtypeStruct((1, 1), jnp.float32),
        ],
        scratch_shapes=[pltpu.SMEM((2,), jnp.float32)],
    )(ea8, We1, ae1, We2, ae2)


def _mid_body(p0_ref, p1_ref, b_ref, w_ref, as_ref, ad_ref,
              hrelu_ref, h_ref, s_ref, d_ref, m_ref, m_scr):
    i = pl.program_id(0)
    xb = p0_ref[...] + p1_ref[...] + b_ref[...]
    xb = jnp.maximum(xb, 0.0)
    hrelu_ref[...] = xb
    h = jnp.dot(xb, w_ref[...], preferred_element_type=jnp.float32)
    h_ref[...] = h
    s = jnp.dot(h, as_ref[...], preferred_element_type=jnp.float32)
    d = jnp.dot(h, ad_ref[...], preferred_element_type=jnp.float32)
    s_ref[...] = s
    d_ref[...] = d
    prev = jnp.where(i == 0, -jnp.inf, m_scr[0])
    m_scr[0] = jnp.maximum(prev, jnp.max(s))

    @pl.when(i == NB - 1)
    def _():
        m_ref[...] = jnp.full((1, 1), m_scr[0], jnp.float32)


def _mid_project(p0, p1, b, W, a_s, a_d):
    return pl.pallas_call(
        _mid_body,
        grid=(NB,),
        in_specs=[
            pl.BlockSpec((RB, D), lambda i: (i, 0)),
            pl.BlockSpec((RB, D), lambda i: (i, 0)),
            pl.BlockSpec((1, D), lambda i: (0, 0)),
            pl.BlockSpec((D, D), lambda i: (0, 0)),
            pl.BlockSpec((D, 1), lambda i: (0, 0)),
            pl.BlockSpec((D, 1), lambda i: (0, 0)),
        ],
        out_specs=[
            pl.BlockSpec((RB, D), lambda i: (i, 0)),
            pl.BlockSpec((RB, D), lambda i: (i, 0)),
            pl.BlockSpec((RB, 1), lambda i: (i, 0)),
            pl.BlockSpec((RB, 1), lambda i: (i, 0)),
            pl.BlockSpec((1, 1), lambda i: (0, 0)),
        ],
        out_shape=[
            jax.ShapeDtypeStruct((N, D), jnp.float32),
            jax.ShapeDtypeStruct((N, D), jnp.float32),
            jax.ShapeDtypeStruct((N, 1), jnp.float32),
            jax.ShapeDtypeStruct((N, 1), jnp.float32),
            jax.ShapeDtypeStruct((1, 1), jnp.float32),
        ],
        scratch_shapes=[pltpu.SMEM((1,), jnp.float32)],
    )(p0, p1, b, W, a_s, a_d)


def _asm_body(p0_ref, p1_ref, b_ref, out_ref):
    out_ref[...] = p0_ref[...] + p1_ref[...] + b_ref[...]


def _assemble(p0, p1, b):
    return pl.pallas_call(
        _asm_body,
        grid=(NB,),
        in_specs=[
            pl.BlockSpec((RB, D), lambda i: (i, 0)),
            pl.BlockSpec((RB, D), lambda i: (i, 0)),
            pl.BlockSpec((1, D), lambda i: (0, 0)),
        ],
        out_specs=pl.BlockSpec((RB, D), lambda i: (i, 0)),
        out_shape=jax.ShapeDtypeStruct((N, D), jnp.float32),
    )(p0, p1, b)


BLK = 2000
VPB = BLK // 16
NBLK_A = EPT // BLK
NBLK_B = EPB // BLK
NSUB = BLK // CHUNK
NR = NP // D


def _rc(i16):
    return jnp.right_shift(i16, 7), jnp.bitwise_and(i16, 127)


def _ex16(v_asrc, v_adst, v_srcc, v_dstc, v_ae, it, scon):
    ds = pl.ds(it * 16, 16)
    s16 = v_srcc[ds]
    d16 = v_dstc[ds]
    ae16 = v_ae[ds]
    dr, dc = _rc(d16)
    asv = plsc.load_gather(v_asrc, list(_rc(s16)))
    adv = plsc.load_gather(v_adst, [dr, dc])
    tot = asv + adv + ae16
    al = jnp.maximum(tot, _NEG_SLOPE * tot)
    sh = adv + scon
    cshift = jnp.maximum(sh, _NEG_SLOPE * sh)
    return dr, dc, jnp.exp(al - cshift)


GRP = 5


def _sc_body(h_hbm, src_h, dst_h, aed_h, asrc_h, adst_h, scon_h,
             out0, out1,
             v_t0, v_t1, v_t2, v_srcc, v_dstc, v_ae,
             v_srcb, v_dstb, v_aeb,
             v_d0, v_d1, v_d2, v_ri, v_sc,
             g0, g1, g2, s0, s1, s2, stA, stB, wb,
             s_den, s_acc):
    cid = lax.axis_index("c")
    sid = lax.axis_index("s")
    ebase = sid * EPT
    rowbufs = [v_t0, v_t1, v_t2]
    dibufs = [v_d0, v_d1, v_d2]
    gsems = [g0, g1, g2]
    ssems = [s0, s1, s2]
    sets = [(v_srcc, v_dstc, v_ae), (v_srcb, v_dstb, v_aeb)]
    stsems = [stA, stB]

    def stage(off, si, srcs):
        for i in range(3):
            pltpu.async_copy(srcs[i].at[pl.ds(off, BLK)],
                             sets[si][i], stsems[si])

    def wait_stage(si, srcs):
        for i in range(3):
            pltpu.make_async_copy(srcs[i].at[pl.ds(0, BLK)],
                                  sets[si][i], stsems[si]).wait()

    pltpu.sync_copy(scon_h, v_sc)
    scon = v_sc[pl.ds(0, 16)][0]

    def zrows(r, _):
        for j in range(D // 16):
            v_t0[r, pl.ds(j * 16, 16)] = jnp.zeros((16,), jnp.float32)
            v_t2[r, pl.ds(j * 16, 16)] = jnp.zeros((16,), jnp.float32)
        return 0
    lax.fori_loop(0, NR, zrows, 0)

    @pl.when(sid == 0)
    def _():
        pltpu.sync_copy(v_t0, s_den)
    for j in range(RPT // NR):
        pltpu.sync_copy(v_t0, s_acc.at[pl.ds(sid * RPT + j * NR, NR), :])

    pltpu.sync_copy(asrc_h, v_t0)
    pltpu.sync_copy(adst_h, v_t1)

    for j in range(NR // 16):
        v_ri[pl.ds(j * 16, 16)] = lax.iota(jnp.int32, 16) + (16 * j)

    srcs_a = (src_h, dst_h, aed_h)

    def a_compute(cur):
        def body(it, _):
            dr, dc, ex = _ex16(v_t0, v_t1, cur[0], cur[1], cur[2], it, scon)
            plsc.addupdate_scatter(v_t2, [dr, dc], ex)
            return 0
        lax.fori_loop(0, VPB, body, 0)

    with jax.named_scope("phA"):
        stage(ebase, 0, srcs_a)

        def a_pair(j, _):
            wait_stage(0, srcs_a)
            stage(ebase + (2 * j + 1) * BLK, 1, srcs_a)
            a_compute(sets[0])
            wait_stage(1, srcs_a)
            off2 = ebase + jnp.where(j == NBLK_A // 2 - 1, 0,
                                     (2 * j + 2)) * BLK
            stage(off2, 0, srcs_a)
            a_compute(sets[1])
            return 0
        lax.fori_loop(0, NBLK_A // 2, a_pair, 0)
        wait_stage(0, srcs_a)

    with jax.named_scope("dred"):
        plsc.subcore_barrier()
        pltpu.sync_copy(v_t2, s_den.at[v_ri], add=True)
        plsc.subcore_barrier()
        pltpu.sync_copy(s_den, v_t2)

    bbase = ebase + cid * EPB

    def coef_compute(cur):
        def coef(it, _):
            dr, dc, ex = _ex16(v_t0, v_t1, cur[0], cur[1], cur[2], it, scon)
            den = plsc.load_gather(v_t2, [dr, dc])
            cur[2][pl.ds(it * 16, 16)] = ex / (den + 1e-16)
            return 0
        lax.fori_loop(0, VPB, coef, 0)

    def b_block(cur):
        def group(g, _, _cur=cur):
                gbase = g * (GRP * CHUNK)
                gd = [None] * GRP
                sd = [None] * GRP

                def fill_di(k):
                    b = k % 3
                    for j in range(CHUNK // 16):
                        dibufs[b][pl.ds(j * 16, 16)] = (
                            _cur[1][pl.ds(gbase + k * CHUNK + j * 16, 16)])

                def issue_gather(k):
                    b = k % 3
                    gd[k] = pltpu.async_copy(
                        h_hbm.at[_cur[0].at[pl.ds(gbase + k * CHUNK, CHUNK)]],
                        rowbufs[b], gsems[b])

                fill_di(0)
                issue_gather(0)
                fill_di(1)
                issue_gather(1)
                for k in range(GRP):
                    b = k % 3
                    gd[k].wait()

                    def scale(r16, _, _b=b, _k=k):
                        cf16 = _cur[2][pl.ds(gbase + _k * CHUNK + r16 * 16,
                                             16)]
                        for jj in range(16):
                            cf = cf16[jj]
                            r = r16 * 16 + jj
                            for j in range(D // 16):
                                ds = pl.ds(j * 16, 16)
                                rowbufs[_b][r, ds] = rowbufs[_b][r, ds] * cf
                        return 0
                    lax.fori_loop(0, CHUNK // 16, scale, 0)
                    sd[k] = pltpu.async_copy(
                        rowbufs[b], s_acc.at[dibufs[b]], ssems[b], add=True)
                    nk = k + 2
                    if nk < GRP:
                        if nk >= 3:
                            sd[nk - 3].wait()
                        fill_di(nk)
                        issue_gather(nk)
                for k in range(max(GRP - 3, 2), GRP):
                    sd[k].wait()
                return 0
        lax.fori_loop(0, NSUB // GRP, group, 0)

    with jax.named_scope("phB"):
        stage(bbase, 0, srcs_a)

        def b_pair(j, _):
            wait_stage(0, srcs_a)
            stage(bbase + (2 * j + 1) * BLK, 1, srcs_a)
            coef_compute(sets[0])
            b_block(sets[0])
            wait_stage(1, srcs_a)
            stage(bbase + (2 * j + 2) * BLK, 0, srcs_a)
            coef_compute(sets[1])
            b_block(sets[1])
            return 0
        lax.fori_loop(0, NBLK_B // 2, b_pair, 0)
        wait_stage(0, srcs_a)
        coef_compute(sets[0])
        b_block(sets[0])

    plsc.subcore_barrier()
    rds = pl.ds(sid * RPT, RPT)

    @pl.when(cid == 0)
    def _():
        pltpu.sync_copy(s_acc.at[rds], out0.at[rds])

    @pl.when(cid == 1)
    def _():
        pltpu.sync_copy(s_acc.at[rds], out1.at[rds])


_sc_layer = pl.kernel(
    _sc_body,
    out_type=(
        jax.ShapeDtypeStruct((NP, D), jnp.float32),
        jax.ShapeDtypeStruct((NP, D), jnp.float32),
    ),
    mesh=plsc.VectorSubcoreMesh(core_axis_name="c", subcore_axis_name="s"),
    compiler_params=pltpu.CompilerParams(needs_layout_passes=False),
    scratch_types=dict(
        v_t0=pltpu.VMEM((NR, D), jnp.float32),
        v_t1=pltpu.VMEM((NR, D), jnp.float32),
        v_t2=pltpu.VMEM((NR, D), jnp.float32),
        v_srcc=pltpu.VMEM((BLK,), jnp.int32),
        v_dstc=pltpu.VMEM((BLK,), jnp.int32),
        v_ae=pltpu.VMEM((BLK,), jnp.float32),
        v_srcb=pltpu.VMEM((BLK,), jnp.int32),
        v_dstb=pltpu.VMEM((BLK,), jnp.int32),
        v_aeb=pltpu.VMEM((BLK,), jnp.float32),
        v_d0=pltpu.VMEM((CHUNK,), jnp.int32),
        v_d1=pltpu.VMEM((CHUNK,), jnp.int32),
        v_d2=pltpu.VMEM((CHUNK,), jnp.int32),
        v_ri=pltpu.VMEM((NR,), jnp.int32),
        v_sc=pltpu.VMEM((16,), jnp.float32),
        g0=pltpu.SemaphoreType.DMA,
        g1=pltpu.SemaphoreType.DMA,
        g2=pltpu.SemaphoreType.DMA,
        s0=pltpu.SemaphoreType.DMA,
        s1=pltpu.SemaphoreType.DMA,
        s2=pltpu.SemaphoreType.DMA,
        stA=pltpu.SemaphoreType.DMA,
        stB=pltpu.SemaphoreType.DMA,
        wb=pltpu.SemaphoreType.DMA,
        s_den=pltpu.VMEM_SHARED((NR, D), jnp.float32),
        s_acc=pltpu.VMEM_SHARED((NP, D), jnp.float32),
    ),
)


def _pad_nodes(a):
    return jnp.pad(a.reshape(N), (0, NP - N)).reshape(NR, D)


def kernel(x, edge_index, edge_attr, W1, We1, as1, ad1, ae1, b1,
           W2, We2, as2, ad2, ae2, b2):
    src = edge_index[0].astype(jnp.int32)
    dst = edge_index[1].astype(jnp.int32)

    h1, asrc1, adst1, m1 = _node_project(
        x, W1, as1.reshape(D, 1), ad1.reshape(D, 1))
    aed1, aed2, me1, me2 = _edge_logits(
        edge_attr.reshape(E8, D), We1, ae1.reshape(D, 1),
        We2, ae2.reshape(D, 1))

    scon1 = jnp.pad((m1 + me1).reshape(1), (0, 15))
    p1_0, p1_1 = _sc_layer(
        h1, src, dst, aed1.reshape(E),
        _pad_nodes(asrc1), _pad_nodes(adst1), scon1)

    h, h2, asrc2, adst2, m2 = _mid_project(
        p1_0, p1_1, b1.reshape(1, D), W2,
        as2.reshape(D, 1), ad2.reshape(D, 1))

    scon2 = jnp.pad((m2 + me2).reshape(1), (0, 15))
    p2_0, p2_1 = _sc_layer(
        h2, src, dst, aed2.reshape(E),
        _pad_nodes(asrc2), _pad_nodes(adst2), scon2)

    x2 = _assemble(p2_0, p2_1, b2.reshape(1, D))
    return (x2, h)
